# baseline (speedup 1.0000x reference)
import functools

import jax
import jax.numpy as jnp
from jax import lax
from jax.experimental import pallas as pl
from jax.experimental.pallas import tpu as pltpu
from jax.experimental.pallas import tpu_sc as plsc

H, S, D = 12, 2048, 64
G2 = 6
W = 2 * D
NC, NS = 2, 16
NW = NC * NS
COLS_PER_W = (H * D) // NW
L = 16
ROWS = 256


def _sort_body(v_ref, sv_ref, p_ref):
    v = v_ref[0]
    sub = lax.broadcasted_iota(jnp.int32, (S, W), 0)
    idx = sub
    k = 2
    while k <= S:
        j = k // 2
        while j >= 1:
            if j >= 4:
                g = S // (2 * j)
                kk = k // (2 * j)
                vr = v.reshape(g, 2, j, W)
                ir = idx.reshape(g, 2, j, W)
                a_v, b_v = vr[:, 0], vr[:, 1]
                a_i, b_i = ir[:, 0], ir[:, 1]
                gid = lax.broadcasted_iota(jnp.int32, (g, 1, 1), 0)
                up = (gid & kk) == 0
                lt = (a_v < b_v) | ((a_v == b_v) & (a_i < b_i))
                first = lt == up
                lo_v = jnp.where(first, a_v, b_v)
                hi_v = jnp.where(first, b_v, a_v)
                lo_i = jnp.where(first, a_i, b_i)
                hi_i = jnp.where(first, b_i, a_i)
                v = jnp.stack([lo_v, hi_v], axis=1).reshape(S, W)
                idx = jnp.stack([lo_i, hi_i], axis=1).reshape(S, W)
            else:
                up = (sub & k) == 0
                low = (sub & j) == 0
                pv = jnp.where(low, jnp.roll(v, -j, axis=0), jnp.roll(v, j, axis=0))
                pi = jnp.where(low, jnp.roll(idx, -j, axis=0), jnp.roll(idx, j, axis=0))
                lt = (v < pv) | ((v == pv) & (idx < pi))
                keep = lt == (up == low)
                v = jnp.where(keep, v, pv)
                idx = jnp.where(keep, idx, pi)
            j //= 2
        k *= 2
    sv_ref[0] = v
    p_ref[0] = idx


_tc_sort = pl.pallas_call(
    _sort_body,
    grid=(G2,),
    in_specs=[pl.BlockSpec((1, S, W), lambda h: (h, 0, 0))],
    out_specs=[
        pl.BlockSpec((1, S, W), lambda h: (h, 0, 0)),
        pl.BlockSpec((1, S, W), lambda h: (h, 0, 0)),
    ],
    out_shape=[
        jax.ShapeDtypeStruct((G2, S, W), jnp.float32),
        jax.ShapeDtypeStruct((G2, S, W), jnp.int32),
    ],
)


def _sc_body(sv_hbm, p_hbm, out_hbm, ai_hbm, sv_v, idx_v, out_v, p0_v, ai_v):
    wid = lax.axis_index("s") * NC + lax.axis_index("c")

    def scatter_perm(dst_v, idx_ref, val_ref):
        def chunk(t, carry):
            idxs = idx_ref[pl.ds(t * L, L)]
            vals = val_ref[pl.ds(t * L, L)]
            plsc.store_scatter(dst_v, [idxs], vals)
            return carry

        lax.fori_loop(0, S // L, chunk, 0)

    base = wid * COLS_PER_W
    for n in range(COLS_PER_W):
        c = base + n
        h = c // D
        d = c % D
        dd = lax.rem(d + 1, D)
        pltpu.sync_copy(sv_hbm.at[h, d], sv_v)
        pltpu.sync_copy(p_hbm.at[h, dd], idx_v)
        scatter_perm(out_v, idx_v, sv_v)
        pltpu.sync_copy(out_v, out_hbm.at[h, d])

    @pl.when(wid < H)
    def _():
        h = wid
        pltpu.sync_copy(p_hbm.at[h, 0], p0_v)
        pltpu.sync_copy(p_hbm.at[h, 1], idx_v)
        scatter_perm(ai_v, idx_v, p0_v)
        pltpu.sync_copy(ai_v, ai_hbm.at[h])


_sc_scatter = functools.partial(
    pl.kernel,
    out_type=(
        jax.ShapeDtypeStruct((H, D, S), jnp.float32),
        jax.ShapeDtypeStruct((H, S), jnp.int32),
    ),
    mesh=plsc.VectorSubcoreMesh(core_axis_name="c", subcore_axis_name="s"),
    compiler_params=pltpu.CompilerParams(needs_layout_passes=False),
    scratch_types=[
        pltpu.VMEM((S,), jnp.float32),
        pltpu.VMEM((S,), jnp.int32),
        pltpu.VMEM((S,), jnp.float32),
        pltpu.VMEM((S,), jnp.int32),
        pltpu.VMEM((S,), jnp.int32),
    ],
)(_sc_body)


def _onehot_body(ai_ref, attn_ref):
    ai = ai_ref[0]
    j = lax.broadcasted_iota(jnp.int32, (ROWS, S), 1)
    attn_ref[0] = (ai == j).astype(jnp.int32)


_tc_onehot = pl.pallas_call(
    _onehot_body,
    grid=(H, S // ROWS),
    in_specs=[pl.BlockSpec((1, ROWS, 1), lambda h, r: (h, r, 0))],
    out_specs=pl.BlockSpec((1, ROWS, S), lambda h, r: (h, r, 0)),
    out_shape=jax.ShapeDtypeStruct((H, S, S), jnp.int32),
)


def kernel(q, k, v):
    del q, k
    vP = v.reshape(G2, 2, S, D).transpose(0, 2, 1, 3).reshape(G2, S, W)
    sv6, p6 = _tc_sort(vP)
    svT = sv6.reshape(G2, S, 2, D).transpose(0, 2, 3, 1).reshape(H, D, S)
    pT = p6.reshape(G2, S, 2, D).transpose(0, 2, 3, 1).reshape(H, D, S)
    outT, ai = _sc_scatter(svT, pT)
    out = jnp.swapaxes(outT, -1, -2).reshape(1, H, S, D)
    attn = _tc_onehot(ai.reshape(H, S, 1)).reshape(1, H, S, S)
    return out, attn


# speedup vs baseline: 1.3809x; 1.3809x over previous
import functools

import jax
import jax.numpy as jnp
from jax import lax
from jax.experimental import pallas as pl
from jax.experimental.pallas import tpu as pltpu
from jax.experimental.pallas import tpu_sc as plsc

H, S, D = 12, 2048, 64
G2 = 6
W = 2 * D
NC, NS = 2, 16
NW = NC * NS
COLS_PER_W = (H * D) // NW
L = 16
ROWS = 256


def _sort_body(v_ref, sv_ref, p_ref):
    v = v_ref[0]
    sub = lax.broadcasted_iota(jnp.int32, (S, W), 0)
    idx = sub
    k = 2
    while k <= S:
        j = k // 2
        while j >= 1:
            if j >= 8:
                g = S // (2 * j)
                kk = k // (2 * j)
                vr = v.reshape(g, 2, j, W)
                ir = idx.reshape(g, 2, j, W)
                a_v, b_v = vr[:, 0], vr[:, 1]
                a_i, b_i = ir[:, 0], ir[:, 1]
                gid = lax.broadcasted_iota(jnp.int32, (g, 1, 1), 0)
                up = (gid & kk) == 0
                lt = (a_v < b_v) | ((a_v == b_v) & (a_i < b_i))
                first = lt == up
                lo_v = jnp.where(first, a_v, b_v)
                hi_v = jnp.where(first, b_v, a_v)
                lo_i = jnp.where(first, a_i, b_i)
                hi_i = jnp.where(first, b_i, a_i)
                v = jnp.stack([lo_v, hi_v], axis=1).reshape(S, W)
                idx = jnp.stack([lo_i, hi_i], axis=1).reshape(S, W)
            else:
                up = (sub & k) == 0
                low = (sub & j) == 0
                pv = jnp.where(low, jnp.roll(v, -j, axis=0), jnp.roll(v, j, axis=0))
                pi = jnp.where(low, jnp.roll(idx, -j, axis=0), jnp.roll(idx, j, axis=0))
                lt = (v < pv) | ((v == pv) & (idx < pi))
                keep = lt == (up == low)
                v = jnp.where(keep, v, pv)
                idx = jnp.where(keep, idx, pi)
            j //= 2
        k *= 2
    sv_ref[0] = v
    p_ref[0] = idx


_tc_sort = pl.pallas_call(
    _sort_body,
    grid=(G2,),
    in_specs=[pl.BlockSpec((1, S, W), lambda h: (h, 0, 0))],
    out_specs=[
        pl.BlockSpec((1, S, W), lambda h: (h, 0, 0)),
        pl.BlockSpec((1, S, W), lambda h: (h, 0, 0)),
    ],
    out_shape=[
        jax.ShapeDtypeStruct((G2, S, W), jnp.float32),
        jax.ShapeDtypeStruct((G2, S, W), jnp.int32),
    ],
)


def _sc_body(sv_hbm, p_hbm, out_hbm, ai_hbm, sv_v, idx_v, out_v, p0_v, ai_v):
    wid = lax.axis_index("s") * NC + lax.axis_index("c")

    def scatter_perm(dst_v, idx_ref, val_ref):
        def chunk(t, carry):
            idxs = idx_ref[pl.ds(t * L, L)]
            vals = val_ref[pl.ds(t * L, L)]
            plsc.store_scatter(dst_v, [idxs], vals)
            return carry

        lax.fori_loop(0, S // L, chunk, 0)

    base = wid * COLS_PER_W
    for n in range(COLS_PER_W):
        c = base + n
        h = c // D
        d = c % D
        dd = lax.rem(d + 1, D)
        pltpu.sync_copy(sv_hbm.at[h, d], sv_v)
        pltpu.sync_copy(p_hbm.at[h, dd], idx_v)
        scatter_perm(out_v, idx_v, sv_v)
        pltpu.sync_copy(out_v, out_hbm.at[h, d])

    @pl.when(wid < H)
    def _():
        h = wid
        pltpu.sync_copy(p_hbm.at[h, 0], p0_v)
        pltpu.sync_copy(p_hbm.at[h, 1], idx_v)
        scatter_perm(ai_v, idx_v, p0_v)
        pltpu.sync_copy(ai_v, ai_hbm.at[h])


_sc_scatter = functools.partial(
    pl.kernel,
    out_type=(
        jax.ShapeDtypeStruct((H, D, S), jnp.float32),
        jax.ShapeDtypeStruct((H, S), jnp.int32),
    ),
    mesh=plsc.VectorSubcoreMesh(core_axis_name="c", subcore_axis_name="s"),
    compiler_params=pltpu.CompilerParams(needs_layout_passes=False),
    scratch_types=[
        pltpu.VMEM((S,), jnp.float32),
        pltpu.VMEM((S,), jnp.int32),
        pltpu.VMEM((S,), jnp.float32),
        pltpu.VMEM((S,), jnp.int32),
        pltpu.VMEM((S,), jnp.int32),
    ],
)(_sc_body)


def _onehot_body(ai_ref, attn_ref):
    ai = ai_ref[0]
    j = lax.broadcasted_iota(jnp.int32, (ROWS, S), 1)
    attn_ref[0] = (ai == j).astype(jnp.int32)


_tc_onehot = pl.pallas_call(
    _onehot_body,
    grid=(H, S // ROWS),
    in_specs=[pl.BlockSpec((1, ROWS, 1), lambda h, r: (h, r, 0))],
    out_specs=pl.BlockSpec((1, ROWS, S), lambda h, r: (h, r, 0)),
    out_shape=jax.ShapeDtypeStruct((H, S, S), jnp.int32),
)


def kernel(q, k, v):
    del q, k
    vP = v.reshape(G2, 2, S, D).transpose(0, 2, 1, 3).reshape(G2, S, W)
    sv6, p6 = _tc_sort(vP)
    svT = sv6.reshape(G2, S, 2, D).transpose(0, 2, 3, 1).reshape(H, D, S)
    pT = p6.reshape(G2, S, 2, D).transpose(0, 2, 3, 1).reshape(H, D, S)
    outT, ai = _sc_scatter(svT, pT)
    out = jnp.swapaxes(outT, -1, -2).reshape(1, H, S, D)
    attn = _tc_onehot(ai.reshape(H, S, 1)).reshape(1, H, S, S)
    return out, attn


# speedup vs baseline: 1.5118x; 1.0948x over previous
import functools

import jax
import jax.numpy as jnp
from jax import lax
from jax.experimental import pallas as pl
from jax.experimental.pallas import tpu as pltpu
from jax.experimental.pallas import tpu_sc as plsc

H, S, D = 12, 2048, 64
G2 = 6
W = 2 * D
NC, NS = 2, 16
NW = NC * NS
COLS_PER_W = (H * D) // NW
L = 16
ROWS = 256


def _sort_body(v_ref, sv_ref, p_ref):
    v = v_ref[0]
    sub = lax.broadcasted_iota(jnp.int32, (S, W), 0)
    idx = sub
    k = 2
    while k <= S:
        j = k // 2
        while j >= 1:
            if j >= 8:
                g = S // (2 * j)
                kk = k // (2 * j)
                vr = v.reshape(g, 2, j, W)
                ir = idx.reshape(g, 2, j, W)
                a_v, b_v = vr[:, 0], vr[:, 1]
                a_i, b_i = ir[:, 0], ir[:, 1]
                gid = lax.broadcasted_iota(jnp.int32, (g, 1, 1), 0)
                up = (gid & kk) == 0
                lt = (a_v < b_v) | ((a_v == b_v) & (a_i < b_i))
                first = lt == up
                lo_v = jnp.where(first, a_v, b_v)
                hi_v = jnp.where(first, b_v, a_v)
                lo_i = jnp.where(first, a_i, b_i)
                hi_i = jnp.where(first, b_i, a_i)
                v = jnp.stack([lo_v, hi_v], axis=1).reshape(S, W)
                idx = jnp.stack([lo_i, hi_i], axis=1).reshape(S, W)
            else:
                up = (sub & k) == 0
                low = (sub & j) == 0
                pv = jnp.where(low, jnp.roll(v, -j, axis=0), jnp.roll(v, j, axis=0))
                pi = jnp.where(low, jnp.roll(idx, -j, axis=0), jnp.roll(idx, j, axis=0))
                lt = (v < pv) | ((v == pv) & (idx < pi))
                keep = lt == (up == low)
                v = jnp.where(keep, v, pv)
                idx = jnp.where(keep, idx, pi)
            j //= 2
        k *= 2
    sv_ref[0] = v
    p_ref[0] = idx


_tc_sort = pl.pallas_call(
    _sort_body,
    grid=(G2,),
    in_specs=[pl.BlockSpec((1, S, W), lambda h: (h, 0, 0))],
    out_specs=[
        pl.BlockSpec((1, S, W), lambda h: (h, 0, 0)),
        pl.BlockSpec((1, S, W), lambda h: (h, 0, 0)),
    ],
    out_shape=[
        jax.ShapeDtypeStruct((G2, S, W), jnp.float32),
        jax.ShapeDtypeStruct((G2, S, W), jnp.int32),
    ],
)


def _scatter_perm(dst_v, idx_ref, val_ref):
    def chunk(t, carry):
        idxs = idx_ref[pl.ds(t * L, L)]
        vals = val_ref[pl.ds(t * L, L)]
        plsc.store_scatter(dst_v, [idxs], vals)
        return carry

    lax.fori_loop(0, S // L, chunk, 0)


def _sc_ai_body(p_hbm, ai_hbm, p0_v, idx_v, ai_v):
    wid = lax.axis_index("s") * NC + lax.axis_index("c")

    @pl.when(wid < H)
    def _():
        h = wid
        pltpu.sync_copy(p_hbm.at[h, 0], p0_v)
        pltpu.sync_copy(p_hbm.at[h, 1], idx_v)
        _scatter_perm(ai_v, idx_v, p0_v)
        pltpu.sync_copy(ai_v, ai_hbm.at[h])


_sc_ai = functools.partial(
    pl.kernel,
    out_type=jax.ShapeDtypeStruct((H, S), jnp.int32),
    mesh=plsc.VectorSubcoreMesh(core_axis_name="c", subcore_axis_name="s"),
    compiler_params=pltpu.CompilerParams(needs_layout_passes=False),
    scratch_types=[
        pltpu.VMEM((S,), jnp.int32),
        pltpu.VMEM((S,), jnp.int32),
        pltpu.VMEM((S,), jnp.int32),
    ],
)(_sc_ai_body)


def _sc_out_body(sv_hbm, p_hbm, out_hbm, sv_v, idx_v, out_v):
    wid = lax.axis_index("s") * NC + lax.axis_index("c")
    base = wid * COLS_PER_W
    for n in range(COLS_PER_W):
        c = base + n
        h = c // D
        d = c % D
        dd = lax.rem(d + 1, D)
        pltpu.sync_copy(sv_hbm.at[h, d], sv_v)
        pltpu.sync_copy(p_hbm.at[h, dd], idx_v)
        _scatter_perm(out_v, idx_v, sv_v)
        pltpu.sync_copy(out_v, out_hbm.at[h, d])


_sc_out = functools.partial(
    pl.kernel,
    out_type=jax.ShapeDtypeStruct((H, D, S), jnp.float32),
    mesh=plsc.VectorSubcoreMesh(core_axis_name="c", subcore_axis_name="s"),
    compiler_params=pltpu.CompilerParams(needs_layout_passes=False),
    scratch_types=[
        pltpu.VMEM((S,), jnp.float32),
        pltpu.VMEM((S,), jnp.int32),
        pltpu.VMEM((S,), jnp.float32),
    ],
)(_sc_out_body)


def _onehot_body(ai_ref, attn_ref):
    ai = ai_ref[0]
    j = lax.broadcasted_iota(jnp.int32, (ROWS, S), 1)
    attn_ref[0] = (ai == j).astype(jnp.int32)


_tc_onehot = pl.pallas_call(
    _onehot_body,
    grid=(H, S // ROWS),
    in_specs=[pl.BlockSpec((1, ROWS, 1), lambda h, r: (h, r, 0))],
    out_specs=pl.BlockSpec((1, ROWS, S), lambda h, r: (h, r, 0)),
    out_shape=jax.ShapeDtypeStruct((H, S, S), jnp.int32),
)


def kernel(q, k, v):
    del q, k
    vP = v.reshape(G2, 2, S, D).transpose(0, 2, 1, 3).reshape(G2, S, W)
    sv6, p6 = _tc_sort(vP)
    svT = sv6.reshape(G2, S, 2, D).transpose(0, 2, 3, 1).reshape(H, D, S)
    pT = p6.reshape(G2, S, 2, D).transpose(0, 2, 3, 1).reshape(H, D, S)
    ai = _sc_ai(pT)
    outT = _sc_out(svT, pT)
    out = jnp.swapaxes(outT, -1, -2).reshape(1, H, S, D)
    attn = _tc_onehot(ai.reshape(H, S, 1)).reshape(1, H, S, S)
    return out, attn
